# baseline (speedup 1.0000x reference)
import functools

import jax
import jax.numpy as jnp
from jax import lax
from jax.experimental import pallas as pl
from jax.experimental.pallas import tpu as pltpu
from jax.experimental.pallas import tpu_sc as plsc

DIM = 128
PK = DIM // 2
MARGIN = 4.0
BATCH = 4096
NROWS = 1000
NC = 2
NS = 16
NW = NC * NS
L = 16
TRIPLES_PER_W = BATCH // NW
GROUPS = TRIPLES_PER_W // L
IDS = 1024


def _rsqrt(x):
    x = jnp.maximum(x, 1e-30)
    i = plsc.bitcast(x, jnp.int32)
    i = 0x5F3759DF - (i >> 1)
    y = plsc.bitcast(i, jnp.float32)
    for _ in range(3):
        y = y * (1.5 - 0.5 * x * y * y)
    return y


def _unpack(w):
    lo = plsc.bitcast(w << 16, jnp.float32)
    hi = plsc.bitcast(w & jnp.int32(-65536), jnp.float32)
    return lo, hi


def _cross_dots(ebuf, wbuf, lanes, ho, to, ro):
    zeros = jnp.zeros((L,), jnp.float32)
    hrow = lanes + ho
    trow = lanes + to
    rrow = lanes + ro

    def body(d, c):
        dcol = (jnp.broadcast_to(d, (L,)).astype(jnp.int32) + lanes) & (DIM - 1)
        h = plsc.load_gather(ebuf, [hrow, dcol])
        t = plsc.load_gather(ebuf, [trow, dcol])
        r, q = _unpack(plsc.load_gather(wbuf, [rrow, dcol]))
        return (
            c[0] + h * r,
            c[1] + h * t,
            c[2] + r * t,
            c[3] + h * q,
            c[4] + t * q,
        )

    return lax.fori_loop(0, DIM, body, (zeros,) * 5)


def _make_sc_kernel():
    mesh = plsc.VectorSubcoreMesh(core_axis_name="c", subcore_axis_name="s")

    @functools.partial(
        pl.kernel,
        mesh=mesh,
        compiler_params=pltpu.CompilerParams(needs_layout_passes=False, use_tc_tiling_on_sc=False),
        out_type=jax.ShapeDtypeStruct((NW, L), jnp.float32),
        scratch_types=(
            [pltpu.VMEM((GROUPS, 4 * L), jnp.int32),
             pltpu.VMEM((GROUPS, 2 * L), jnp.int32),
             pltpu.VMEM((4 * L, DIM), jnp.float32),
             pltpu.VMEM((4 * L, DIM), jnp.float32),
             pltpu.VMEM((4 * L, DIM), jnp.float32),
             pltpu.VMEM((4 * L, DIM), jnp.float32),
             pltpu.VMEM((2 * L, DIM), jnp.int32),
             pltpu.VMEM((2 * L, DIM), jnp.int32),
             pltpu.VMEM((2 * L, DIM), jnp.int32),
             pltpu.VMEM((2 * L, DIM), jnp.int32),
             pltpu.VMEM((4 * L, DIM), jnp.float32),
             pltpu.VMEM((4 * L, DIM), jnp.float32)]
            + [pltpu.VMEM((4 * L,), jnp.float32) for _ in range(5)]
            + [pltpu.VMEM((IDS,), jnp.float32) for _ in range(5)]
            + [pltpu.VMEM_SHARED((IDS,), jnp.float32) for _ in range(5)]
            + [pltpu.VMEM((L,), jnp.float32),
               pltpu.SemaphoreType.DMA,
               pltpu.SemaphoreType.DMA,
               pltpu.SemaphoreType.DMA,
               pltpu.SemaphoreType.DMA,
               pltpu.SemaphoreType.DMA]
        ),
    )
    def sc_kernel(ent_emb, ent_tr, rel_emb, rel_tr, wrel,
                  eidx_hbm, ridx_hbm, out_hbm, eidx_v, ridx_v,
                  ebA, ebB, ebC, ebD, wrA, wrB, wrC, wrD, ta, tb,
                  loc0, loc1, loc2, loc3, loc4,
                  pre0, pre1, pre2, pre3, pre4,
                  sh0, sh1, sh2, sh3, sh4, acc_v,
                  semA, semB, semC, semD, semP):
        cid = lax.axis_index("c")
        sid = lax.axis_index("s")
        wid = sid * NC + cid
        lanes = lax.iota(jnp.int32, L)

        pltpu.sync_copy(eidx_hbm.at[wid], eidx_v)
        pltpu.sync_copy(ridx_hbm.at[wid], ridx_v)

        sets = ((ebA, wrA, semA), (ebB, wrB, semB),
                (ebC, wrC, semC), (ebD, wrD, semD))

        def fire(g, s):
            eb, wr, sem = s
            pltpu.async_copy(ent_emb.at[eidx_v.at[g]], eb, sem)
            pltpu.async_copy(wrel.at[ridx_v.at[g]], wr, sem)

        def drain(s):
            eb, wr, sem = s
            pltpu.make_async_copy(ent_emb.at[eidx_v.at[0]], eb, sem).wait()
            pltpu.make_async_copy(wrel.at[ridx_v.at[0]], wr, sem).wait()

        fire(0, sets[0])
        fire(1, sets[1])
        fire(2, sets[2])

        base_e = pl.multiple_of(sid * (4 * L), 4 * L)
        base_r = pl.multiple_of(jnp.minimum(base_e, NROWS - 4 * L), 8)
        c1 = pltpu.async_copy(ent_emb.at[pl.ds(base_e, 4 * L)], ta, semP)
        c2 = pltpu.async_copy(ent_tr.at[pl.ds(base_e, 4 * L)], tb, semP)
        c1.wait()
        c2.wait()

        def diag_dots(sub, nacc):
            rows = lanes + sub * L

            def body(d, c):
                dcol = (jnp.broadcast_to(d, (L,)).astype(jnp.int32)
                        + lanes) & (DIM - 1)
                a = plsc.load_gather(ta, [rows, dcol])
                b = plsc.load_gather(tb, [rows, dcol])
                out = (c[0] + a * a, c[1] + a * b)
                if nacc == 3:
                    out = out + (c[2] + b * b,)
                return out

            return lax.fori_loop(0, DIM, body,
                                 (jnp.zeros((L,), jnp.float32),) * nacc)

        for sub in range(4):
            ee, eet = diag_dots(sub, 2)
            loc0[pl.ds(sub * L, L)] = ee
            loc1[pl.ds(sub * L, L)] = eet

        c1 = pltpu.async_copy(rel_emb.at[pl.ds(base_r, 4 * L)], ta, semP)
        c2 = pltpu.async_copy(rel_tr.at[pl.ds(base_r, 4 * L)], tb, semP)
        c1.wait()
        c2.wait()
        for sub in range(4):
            rr, rrt, qq = diag_dots(sub, 3)
            loc2[pl.ds(sub * L, L)] = rr
            loc3[pl.ds(sub * L, L)] = rrt
            loc4[pl.ds(sub * L, L)] = qq

        pltpu.sync_copy(loc0, sh0.at[pl.ds(base_e, 4 * L)])
        pltpu.sync_copy(loc1, sh1.at[pl.ds(base_e, 4 * L)])
        pltpu.sync_copy(loc2, sh2.at[pl.ds(base_r, 4 * L)])
        pltpu.sync_copy(loc3, sh3.at[pl.ds(base_r, 4 * L)])
        pltpu.sync_copy(loc4, sh4.at[pl.ds(base_r, 4 * L)])
        plsc.subcore_barrier()
        for shq, preq in ((sh0, pre0), (sh1, pre1), (sh2, pre2),
                          (sh3, pre3), (sh4, pre4)):
            pltpu.sync_copy(shq, preq)

        def distance(s, g, ho, to, ro):
            eb, wr, _ = s
            h_ids = eidx_v[g, pl.ds(ho * L, L)]
            t_ids = eidx_v[g, pl.ds(to * L, L)]
            r_ids = ridx_v[g, pl.ds(ro * L, L)]
            hh = plsc.load_gather(pre0, [h_ids])
            tt2 = plsc.load_gather(pre0, [t_ids])
            sh = plsc.load_gather(pre1, [h_ids])
            st = plsc.load_gather(pre1, [t_ids])
            rr = plsc.load_gather(pre2, [r_ids])
            rrt = plsc.load_gather(pre3, [r_ids])
            rtrt = plsc.load_gather(pre4, [r_ids])
            hr, ht_d, rt_d, hrt, trt = _cross_dots(
                eb, wr, lanes, ho * L, to * L, ro * L)
            nh2 = hh + 2.0 * sh * hrt + sh * sh * rtrt
            nt2 = tt2 + 2.0 * st * trt + st * st * rtrt
            hp_r = hr + sh * rrt
            hp_tp = ht_d + st * hrt + sh * trt + sh * st * rtrt
            r_tp = rt_d + st * rrt
            inh = _rsqrt(nh2)
            int_ = _rsqrt(nt2)
            inr = _rsqrt(rr)
            d2 = 3.0 + 2.0 * (hp_r * inh * inr - hp_tp * inh * int_
                              - r_tp * inr * int_)
            d2 = jnp.maximum(d2, 0.0)
            return d2 * _rsqrt(d2)

        def compute(s, g, acc):
            pos = distance(s, g, 0, 1, 0)
            neg = distance(s, g, 2, 3, 1)
            return acc + jnp.maximum(pos - neg + MARGIN, 0.0)

        def quad(k, acc):
            for j in range(4):
                g = 4 * k + j
                drain(sets[j])
                acc = compute(sets[j], g, acc)
                nxt = g + 3

                @pl.when(nxt < GROUPS)
                def _(nxt=nxt, j=j):
                    fire(nxt, sets[(j + 3) % 4])

            return acc

        acc = lax.fori_loop(0, GROUPS // 4, quad, jnp.zeros((L,), jnp.float32))
        acc_v[...] = acc
        pltpu.sync_copy(acc_v, out_hbm.at[wid])

    return sc_kernel


_SC_KERNEL = _make_sc_kernel()


def _pair_tables(a, b):
    au = lax.bitcast_convert_type(a, jnp.uint32) & jnp.uint32(0xFFFF0000)
    bu = lax.bitcast_convert_type(b, jnp.uint32) >> 16
    return lax.bitcast_convert_type(au | bu, jnp.int32)


@jax.jit
def kernel(current_triples, corrupted_triples, ent_embedding, rel_embedding,
           ent_transfer, rel_transfer):
    cur = current_triples.astype(jnp.int32)
    cor = corrupted_triples.astype(jnp.int32)

    def wg(col_arrays):
        parts = [a.reshape(NW, GROUPS, L) for a in col_arrays]
        return jnp.stack(parts, axis=2).reshape(NW, GROUPS, len(parts) * L)

    eidx = wg([cur[:, 0], cur[:, 2], cor[:, 0], cor[:, 2]])
    ridx = wg([cur[:, 1], cor[:, 1]])
    wrel = _pair_tables(rel_transfer, rel_embedding)
    partials = _SC_KERNEL(ent_embedding, ent_transfer, rel_embedding,
                          rel_transfer, wrel, eidx, ridx)
    return jnp.sum(partials) / BATCH


# speedup vs baseline: 1.0103x; 1.0103x over previous
import functools

import jax
import jax.numpy as jnp
from jax import lax
from jax.experimental import pallas as pl
from jax.experimental.pallas import tpu as pltpu
from jax.experimental.pallas import tpu_sc as plsc

DIM = 128
PK = DIM // 2
MARGIN = 4.0
BATCH = 4096
NROWS = 1000
NC = 2
NS = 16
NW = NC * NS
L = 16
TRIPLES_PER_W = BATCH // NW
GROUPS = TRIPLES_PER_W // L
IDS = 1024


def _rsqrt(x):
    x = jnp.maximum(x, 1e-30)
    i = plsc.bitcast(x, jnp.int32)
    i = 0x5F3759DF - (i >> 1)
    y = plsc.bitcast(i, jnp.float32)
    for _ in range(3):
        y = y * (1.5 - 0.5 * x * y * y)
    return y


def _unpack(w):
    lo = plsc.bitcast(w << 16, jnp.float32)
    hi = plsc.bitcast(w & jnp.int32(-65536), jnp.float32)
    return lo, hi


def _cross_dots(ebuf, wbuf, lanes, ho, to, ro):
    zeros = jnp.zeros((L,), jnp.float32)
    hrow = lanes + ho
    trow = lanes + to
    rrow = lanes + ro

    def body(d, c):
        dcol = (jnp.broadcast_to(d, (L,)).astype(jnp.int32) + lanes) & (DIM - 1)
        h = plsc.load_gather(ebuf, [hrow, dcol])
        t = plsc.load_gather(ebuf, [trow, dcol])
        r, q = _unpack(plsc.load_gather(wbuf, [rrow, dcol]))
        return (
            c[0] + h * r,
            c[1] + h * t,
            c[2] + r * t,
            c[3] + h * q,
            c[4] + t * q,
        )

    return lax.fori_loop(0, DIM, body, (zeros,) * 5)


def _make_sc_kernel():
    mesh = plsc.VectorSubcoreMesh(core_axis_name="c", subcore_axis_name="s")

    @functools.partial(
        pl.kernel,
        mesh=mesh,
        compiler_params=pltpu.CompilerParams(needs_layout_passes=False, use_tc_tiling_on_sc=False),
        out_type=jax.ShapeDtypeStruct((NW, L), jnp.float32),
        scratch_types=(
            [pltpu.VMEM((GROUPS, 4 * L), jnp.int32),
             pltpu.VMEM((GROUPS, 2 * L), jnp.int32),
             pltpu.VMEM((4 * L, DIM), jnp.float32),
             pltpu.VMEM((4 * L, DIM), jnp.float32),
             pltpu.VMEM((4 * L, DIM), jnp.float32),
             pltpu.VMEM((4 * L, DIM), jnp.float32),
             pltpu.VMEM((2 * L, DIM), jnp.int32),
             pltpu.VMEM((2 * L, DIM), jnp.int32),
             pltpu.VMEM((2 * L, DIM), jnp.int32),
             pltpu.VMEM((2 * L, DIM), jnp.int32),
             pltpu.VMEM((4 * L, DIM), jnp.float32),
             pltpu.VMEM((4 * L, DIM), jnp.float32),
             pltpu.VMEM((4 * L, DIM), jnp.float32),
             pltpu.VMEM((4 * L, DIM), jnp.float32)]
            + [pltpu.VMEM((4 * L,), jnp.float32) for _ in range(5)]
            + [pltpu.VMEM((IDS,), jnp.float32) for _ in range(5)]
            + [pltpu.VMEM_SHARED((IDS,), jnp.float32) for _ in range(5)]
            + [pltpu.VMEM((L,), jnp.float32),
               pltpu.SemaphoreType.DMA,
               pltpu.SemaphoreType.DMA,
               pltpu.SemaphoreType.DMA,
               pltpu.SemaphoreType.DMA,
               pltpu.SemaphoreType.DMA]
        ),
    )
    def sc_kernel(ent_emb, ent_tr, rel_emb, rel_tr, wrel,
                  eidx_hbm, ridx_hbm, out_hbm, eidx_v, ridx_v,
                  ebA, ebB, ebC, ebD, wrA, wrB, wrC, wrD, ta, tb, tc, td,
                  loc0, loc1, loc2, loc3, loc4,
                  pre0, pre1, pre2, pre3, pre4,
                  sh0, sh1, sh2, sh3, sh4, acc_v,
                  semA, semB, semC, semD, semP):
        cid = lax.axis_index("c")
        sid = lax.axis_index("s")
        wid = sid * NC + cid
        lanes = lax.iota(jnp.int32, L)

        pltpu.sync_copy(eidx_hbm.at[wid], eidx_v)
        pltpu.sync_copy(ridx_hbm.at[wid], ridx_v)

        sets = ((ebA, wrA, semA), (ebB, wrB, semB),
                (ebC, wrC, semC), (ebD, wrD, semD))

        def fire(g, s):
            eb, wr, sem = s
            pltpu.async_copy(ent_emb.at[eidx_v.at[g]], eb, sem)
            pltpu.async_copy(wrel.at[ridx_v.at[g]], wr, sem)

        def drain(s):
            eb, wr, sem = s
            pltpu.make_async_copy(ent_emb.at[eidx_v.at[0]], eb, sem).wait()
            pltpu.make_async_copy(wrel.at[ridx_v.at[0]], wr, sem).wait()

        base_e = pl.multiple_of(sid * (4 * L), 4 * L)
        base_r = pl.multiple_of(jnp.minimum(base_e, NROWS - 4 * L), 8)
        c1 = pltpu.async_copy(ent_emb.at[pl.ds(base_e, 4 * L)], ta, semP)
        c2 = pltpu.async_copy(ent_tr.at[pl.ds(base_e, 4 * L)], tb, semP)
        c3 = pltpu.async_copy(rel_emb.at[pl.ds(base_r, 4 * L)], tc, semP)
        c4 = pltpu.async_copy(rel_tr.at[pl.ds(base_r, 4 * L)], td, semP)

        fire(0, sets[0])
        fire(1, sets[1])
        fire(2, sets[2])

        c1.wait()
        c2.wait()
        c3.wait()
        c4.wait()

        def diag_dots(ba, bb, sub, nacc):
            rows = lanes + sub * L

            def body(d, c):
                dcol = (jnp.broadcast_to(d, (L,)).astype(jnp.int32)
                        + lanes) & (DIM - 1)
                a = plsc.load_gather(ba, [rows, dcol])
                b = plsc.load_gather(bb, [rows, dcol])
                out = (c[0] + a * a, c[1] + a * b)
                if nacc == 3:
                    out = out + (c[2] + b * b,)
                return out

            return lax.fori_loop(0, DIM, body,
                                 (jnp.zeros((L,), jnp.float32),) * nacc)

        for sub in range(4):
            ee, eet = diag_dots(ta, tb, sub, 2)
            loc0[pl.ds(sub * L, L)] = ee
            loc1[pl.ds(sub * L, L)] = eet

        for sub in range(4):
            rr, rrt, qq = diag_dots(tc, td, sub, 3)
            loc2[pl.ds(sub * L, L)] = rr
            loc3[pl.ds(sub * L, L)] = rrt
            loc4[pl.ds(sub * L, L)] = qq

        pltpu.sync_copy(loc0, sh0.at[pl.ds(base_e, 4 * L)])
        pltpu.sync_copy(loc1, sh1.at[pl.ds(base_e, 4 * L)])
        pltpu.sync_copy(loc2, sh2.at[pl.ds(base_r, 4 * L)])
        pltpu.sync_copy(loc3, sh3.at[pl.ds(base_r, 4 * L)])
        pltpu.sync_copy(loc4, sh4.at[pl.ds(base_r, 4 * L)])
        plsc.subcore_barrier()
        for shq, preq in ((sh0, pre0), (sh1, pre1), (sh2, pre2),
                          (sh3, pre3), (sh4, pre4)):
            pltpu.sync_copy(shq, preq)

        def distance(s, g, ho, to, ro):
            eb, wr, _ = s
            h_ids = eidx_v[g, pl.ds(ho * L, L)]
            t_ids = eidx_v[g, pl.ds(to * L, L)]
            r_ids = ridx_v[g, pl.ds(ro * L, L)]
            hh = plsc.load_gather(pre0, [h_ids])
            tt2 = plsc.load_gather(pre0, [t_ids])
            sh = plsc.load_gather(pre1, [h_ids])
            st = plsc.load_gather(pre1, [t_ids])
            rr = plsc.load_gather(pre2, [r_ids])
            rrt = plsc.load_gather(pre3, [r_ids])
            rtrt = plsc.load_gather(pre4, [r_ids])
            hr, ht_d, rt_d, hrt, trt = _cross_dots(
                eb, wr, lanes, ho * L, to * L, ro * L)
            nh2 = hh + 2.0 * sh * hrt + sh * sh * rtrt
            nt2 = tt2 + 2.0 * st * trt + st * st * rtrt
            hp_r = hr + sh * rrt
            hp_tp = ht_d + st * hrt + sh * trt + sh * st * rtrt
            r_tp = rt_d + st * rrt
            inh = _rsqrt(nh2)
            int_ = _rsqrt(nt2)
            inr = _rsqrt(rr)
            d2 = 3.0 + 2.0 * (hp_r * inh * inr - hp_tp * inh * int_
                              - r_tp * inr * int_)
            d2 = jnp.maximum(d2, 0.0)
            return d2 * _rsqrt(d2)

        def compute(s, g, acc):
            pos = distance(s, g, 0, 1, 0)
            neg = distance(s, g, 2, 3, 1)
            return acc + jnp.maximum(pos - neg + MARGIN, 0.0)

        def quad(k, acc):
            for j in range(4):
                g = 4 * k + j
                drain(sets[j])
                acc = compute(sets[j], g, acc)
                nxt = g + 3

                @pl.when(nxt < GROUPS)
                def _(nxt=nxt, j=j):
                    fire(nxt, sets[(j + 3) % 4])

            return acc

        acc = lax.fori_loop(0, GROUPS // 4, quad, jnp.zeros((L,), jnp.float32))
        acc_v[...] = acc
        pltpu.sync_copy(acc_v, out_hbm.at[wid])

    return sc_kernel


_SC_KERNEL = _make_sc_kernel()


def _pair_tables(a, b):
    au = lax.bitcast_convert_type(a, jnp.uint32) & jnp.uint32(0xFFFF0000)
    bu = lax.bitcast_convert_type(b, jnp.uint32) >> 16
    return lax.bitcast_convert_type(au | bu, jnp.int32)


@jax.jit
def kernel(current_triples, corrupted_triples, ent_embedding, rel_embedding,
           ent_transfer, rel_transfer):
    cur = current_triples.astype(jnp.int32)
    cor = corrupted_triples.astype(jnp.int32)

    def wg(col_arrays):
        parts = [a.reshape(NW, GROUPS, L) for a in col_arrays]
        return jnp.stack(parts, axis=2).reshape(NW, GROUPS, len(parts) * L)

    eidx = wg([cur[:, 0], cur[:, 2], cor[:, 0], cor[:, 2]])
    ridx = wg([cur[:, 1], cor[:, 1]])
    wrel = _pair_tables(rel_transfer, rel_embedding)
    partials = _SC_KERNEL(ent_embedding, ent_transfer, rel_embedding,
                          rel_transfer, wrel, eidx, ridx)
    return jnp.sum(partials) / BATCH


# speedup vs baseline: 1.1234x; 1.1119x over previous
import functools

import jax
import jax.numpy as jnp
from jax import lax
from jax.experimental import pallas as pl
from jax.experimental.pallas import tpu as pltpu
from jax.experimental.pallas import tpu_sc as plsc

DIM = 128
MARGIN = 4.0
BATCH = 4096
NC = 2
NS = 16
NW = NC * NS
L = 16
TRIPLES_PER_W = BATCH // NW
GROUPS = TRIPLES_PER_W // L


def _rsqrt(x):
    x = jnp.maximum(x, 1e-30)
    i = plsc.bitcast(x, jnp.int32)
    i = 0x5F3759DF - (i >> 1)
    y = plsc.bitcast(i, jnp.float32)
    for _ in range(3):
        y = y * (1.5 - 0.5 * x * y * y)
    return y


def _distance(ebuf, etbuf, rbuf, rtbuf, lanes, ho, to, ro):
    zeros = jnp.zeros((L,), jnp.float32)
    hrow = lanes + ho
    trow = lanes + to
    rrow = lanes + ro

    def body(d, c):
        dcol = (jnp.broadcast_to(d, (L,)).astype(jnp.int32) + lanes) & (DIM - 1)
        h = plsc.load_gather(ebuf, [hrow, dcol])
        t = plsc.load_gather(ebuf, [trow, dcol])
        ht = plsc.load_gather(etbuf, [hrow, dcol])
        tt = plsc.load_gather(etbuf, [trow, dcol])
        r = plsc.load_gather(rbuf, [rrow, dcol])
        rt = plsc.load_gather(rtbuf, [rrow, dcol])
        return (
            c[0] + h * ht,
            c[1] + t * tt,
            c[2] + h * r,
            c[3] + h * t,
            c[4] + r * t,
            c[5] + h * rt,
            c[6] + t * rt,
            c[7] + r * rt,
            c[8] + rt * rt,
            c[9] + r * r,
            c[10] + h * h,
            c[11] + t * t,
        )

    (sh, st, hr, ht_d, rt_d, hrt, trt, rrt, rtrt, rr, hh, tt2) = lax.fori_loop(
        0, DIM, body, (zeros,) * 12)

    nh2 = hh + 2.0 * sh * hrt + sh * sh * rtrt
    nt2 = tt2 + 2.0 * st * trt + st * st * rtrt
    hp_r = hr + sh * rrt
    hp_tp = ht_d + st * hrt + sh * trt + sh * st * rtrt
    r_tp = rt_d + st * rrt
    inh = _rsqrt(nh2)
    int_ = _rsqrt(nt2)
    inr = _rsqrt(rr)
    d2 = 3.0 + 2.0 * (hp_r * inh * inr - hp_tp * inh * int_ - r_tp * inr * int_)
    d2 = jnp.maximum(d2, 0.0)
    return d2 * _rsqrt(d2)


def _make_sc_kernel():
    mesh = plsc.VectorSubcoreMesh(core_axis_name="c", subcore_axis_name="s")

    @functools.partial(
        pl.kernel,
        mesh=mesh,
        compiler_params=pltpu.CompilerParams(needs_layout_passes=False),
        out_type=jax.ShapeDtypeStruct((NW, L), jnp.float32),
        scratch_types=(
            [pltpu.VMEM((GROUPS, 4 * L), jnp.int32),
             pltpu.VMEM((GROUPS, 2 * L), jnp.int32)]
            + [pltpu.VMEM((4 * L, DIM), jnp.float32) for _ in range(4)]
            + [pltpu.VMEM((2 * L, DIM), jnp.float32) for _ in range(4)]
            + [pltpu.VMEM((L,), jnp.float32),
               pltpu.SemaphoreType.DMA, pltpu.SemaphoreType.DMA]
        ),
    )
    def sc_kernel(ent_emb, rel_emb, ent_tr, rel_tr, eidx_hbm, ridx_hbm,
                  out_hbm, eidx_v, ridx_v, ebA, ebB, etA, etB,
                  rbA, rbB, rtA, rtB, acc_v, semA, semB):
        wid = lax.axis_index("s") * NC + lax.axis_index("c")
        pltpu.sync_copy(eidx_hbm.at[wid], eidx_v)
        pltpu.sync_copy(ridx_hbm.at[wid], ridx_v)
        lanes = lax.iota(jnp.int32, L)
        sets = ((ebA, etA, rbA, rtA, semA), (ebB, etB, rbB, rtB, semB))

        def fire(g, s):
            eb, et, rb, rt, sem = s
            pltpu.async_copy(ent_emb.at[eidx_v.at[g]], eb, sem)
            pltpu.async_copy(ent_tr.at[eidx_v.at[g]], et, sem)
            pltpu.async_copy(rel_emb.at[ridx_v.at[g]], rb, sem)
            pltpu.async_copy(rel_tr.at[ridx_v.at[g]], rt, sem)

        def drain(s):
            eb, et, rb, rt, sem = s
            pltpu.make_async_copy(ent_emb.at[eidx_v.at[0]], eb, sem).wait()
            pltpu.make_async_copy(ent_tr.at[eidx_v.at[0]], et, sem).wait()
            pltpu.make_async_copy(rel_emb.at[ridx_v.at[0]], rb, sem).wait()
            pltpu.make_async_copy(rel_tr.at[ridx_v.at[0]], rt, sem).wait()

        def compute(s, acc):
            eb, et, rb, rt, _ = s
            pos = _distance(eb, et, rb, rt, lanes, 0, L, 0)
            neg = _distance(eb, et, rb, rt, lanes, 2 * L, 3 * L, L)
            return acc + jnp.maximum(pos - neg + MARGIN, 0.0)

        fire(0, sets[0])

        def pair(gg, acc):
            fire(2 * gg + 1, sets[1])
            drain(sets[0])
            acc = compute(sets[0], acc)

            @pl.when(gg < GROUPS // 2 - 1)
            def _():
                fire(2 * gg + 2, sets[0])

            drain(sets[1])
            return compute(sets[1], acc)

        acc = lax.fori_loop(0, GROUPS // 2, pair, jnp.zeros((L,), jnp.float32))
        acc_v[...] = acc
        pltpu.sync_copy(acc_v, out_hbm.at[wid])

    return sc_kernel


_SC_KERNEL = _make_sc_kernel()


@jax.jit
def kernel(current_triples, corrupted_triples, ent_embedding, rel_embedding,
           ent_transfer, rel_transfer):
    cur = current_triples.astype(jnp.int32)
    cor = corrupted_triples.astype(jnp.int32)
    def wg(col_arrays):
        parts = [a.reshape(NW, GROUPS, L) for a in col_arrays]
        return jnp.stack(parts, axis=2).reshape(NW, GROUPS, len(parts) * L)

    eidx = wg([cur[:, 0], cur[:, 2], cor[:, 0], cor[:, 2]])
    ridx = wg([cur[:, 1], cor[:, 1]])
    partials = _SC_KERNEL(ent_embedding, rel_embedding, ent_transfer,
                          rel_transfer, eidx, ridx)
    return jnp.sum(partials) / BATCH
